# baseline (speedup 1.0000x reference)
import functools

import jax
import jax.numpy as jnp
from jax import lax
from jax.experimental import pallas as pl
from jax.experimental.pallas import tpu as pltpu
from jax.experimental.pallas import tpu_sc as plsc

_NC = 2
_NS = 16
_NW = _NC * _NS


def _gather3_body(b_per_w, d0, d1, d2,
                  t0, i0, t1, i1, t2, i2,
                  o0, o1, o2,
                  idx0, idx1, idx2,
                  r0, r1, r2,
                  sem):
    wid = lax.axis_index("s") * _NC + lax.axis_index("c")
    base = wid * b_per_w
    pltpu.sync_copy(i0.at[pl.ds(base, b_per_w)], idx0)
    pltpu.sync_copy(i1.at[pl.ds(base, b_per_w)], idx1)
    pltpu.sync_copy(i2.at[pl.ds(base, b_per_w)], idx2)
    c0 = pltpu.async_copy(t0.at[idx0], r0, sem)
    c1 = pltpu.async_copy(t1.at[idx1], r1, sem)
    c2 = pltpu.async_copy(t2.at[idx2], r2, sem)
    c0.wait()
    c1.wait()
    c2.wait()
    pltpu.sync_copy(r0, o0.at[pl.ds(base, b_per_w)])
    pltpu.sync_copy(r1, o1.at[pl.ds(base, b_per_w)])
    pltpu.sync_copy(r2, o2.at[pl.ds(base, b_per_w)])


def _make_gather3(B, d0, d1, d2):
    b_per_w = B // _NW
    mesh = plsc.VectorSubcoreMesh(core_axis_name="c", subcore_axis_name="s")
    return pl.kernel(
        functools.partial(_gather3_body, b_per_w, d0, d1, d2),
        out_type=(
            jax.ShapeDtypeStruct((B, d0), jnp.float32),
            jax.ShapeDtypeStruct((B, d1), jnp.float32),
            jax.ShapeDtypeStruct((B, d2), jnp.float32),
        ),
        mesh=mesh,
        scratch_types=[
            pltpu.VMEM((b_per_w,), jnp.int32),
            pltpu.VMEM((b_per_w,), jnp.int32),
            pltpu.VMEM((b_per_w,), jnp.int32),
            pltpu.VMEM((b_per_w, d0), jnp.float32),
            pltpu.VMEM((b_per_w, d1), jnp.float32),
            pltpu.VMEM((b_per_w, d2), jnp.float32),
            pltpu.SemaphoreType.DMA,
        ],
        compiler_params=pltpu.CompilerParams(use_tc_tiling_on_sc=False),
    )


def _mlp_body(x_ref, w_ref, b_ref, o_ref):
    x = jnp.maximum(x_ref[...], 0.0)
    acc = jax.lax.dot_general(
        x, w_ref[...], (((1,), (0,)), ((), ())),
        preferred_element_type=jnp.float32)
    o_ref[...] = jnp.maximum(acc + b_ref[...], 0.0)


def _mlp(x, w, b):
    B, K = x.shape
    N = w.shape[1]
    BLK = 2048
    return pl.pallas_call(
        _mlp_body,
        grid=(B // BLK,),
        in_specs=[
            pl.BlockSpec((BLK, K), lambda i: (i, 0)),
            pl.BlockSpec((K, N), lambda i: (0, 0)),
            pl.BlockSpec((N,), lambda i: (0,)),
        ],
        out_specs=pl.BlockSpec((BLK, N), lambda i: (i, 0)),
        out_shape=jax.ShapeDtypeStruct((B, N), jnp.float32),
    )(x, w, b)


def kernel(arg0_1, arg1_1, arg2_1, arg3_1, arg4_1, arg5_1, arg6_1, arg7_1, arg8_1):
    B = arg1_1.shape[0]
    g = _make_gather3(B, arg0_1.shape[1], arg2_1.shape[1], arg4_1.shape[1])
    squeeze, squeeze_1, squeeze_2 = g(arg0_1, arg1_1, arg2_1, arg3_1,
                                      arg4_1, arg5_1)
    relu_1 = _mlp(arg7_1, arg6_1, arg8_1)
    return (squeeze, squeeze_1, squeeze_2, arg6_1, relu_1)
